# Initial kernel scaffold; baseline (speedup 1.0000x reference)
import jax
import jax.numpy as jnp
from jax.experimental import pallas as pl


def kernel(tokens, wpe):
    raise NotImplementedError("write your pallas kernel here")


# speedup vs baseline: 5.8030x; 5.8030x over previous
import jax
import jax.numpy as jnp
from jax.experimental import pallas as pl

BSZ = 4
SEQ_LEN = 8192
EMBED_DIM = 768
ROW_BLOCK = 1024


def _bcast_kernel(wpe_ref, out_ref):
    out_ref[...] = jnp.broadcast_to(
        wpe_ref[...][None], (BSZ, ROW_BLOCK, EMBED_DIM)
    )


def kernel(tokens, wpe):
    del tokens
    num_blocks = SEQ_LEN // ROW_BLOCK
    return pl.pallas_call(
        _bcast_kernel,
        grid=(num_blocks,),
        in_specs=[
            pl.BlockSpec((ROW_BLOCK, EMBED_DIM), lambda i: (i, 0)),
        ],
        out_specs=pl.BlockSpec(
            (BSZ, ROW_BLOCK, EMBED_DIM), lambda i: (0, i, 0)
        ),
        out_shape=jax.ShapeDtypeStruct((BSZ, SEQ_LEN, EMBED_DIM), wpe.dtype),
    )(wpe)
